# baseline (speedup 1.0000x reference)
import functools

import jax
import jax.numpy as jnp
from jax import lax
from jax.experimental import pallas as pl
from jax.experimental.pallas import tpu as pltpu
from jax.experimental.pallas import tpu_sc as plsc

NC = 2
NS = 16
NW = NC * NS
CHUNK = 128
NBUF = 2


def _deg_kernel(n_pad, ch, d):
    stripe = n_pad // NS
    mesh = plsc.VectorSubcoreMesh(core_axis_name="c", subcore_axis_name="s", num_cores=NC, num_subcores=NS)

    @functools.partial(
        pl.kernel,
        out_type=jax.ShapeDtypeStruct((NC * n_pad, d), jnp.float32),
        mesh=mesh,
        scratch_types=[
            pltpu.VMEM((ch, CHUNK), jnp.int32),
            pltpu.VMEM((CHUNK, d), jnp.float32),
            pltpu.VMEM_SHARED((n_pad, d), jnp.float32),
            pltpu.SemaphoreType.DMA,
        ],
    )
    def deg_kernel(dst_hbm, ones_hbm, z16_hbm, out_hbm, dst_v, ones_v, acc,
                   ssem):
        c = lax.axis_index("c")
        s = lax.axis_index("s")
        wid = c * NS + s
        pltpu.sync_copy(dst_hbm.at[wid], dst_v)
        pltpu.sync_copy(ones_hbm, ones_v)
        pltpu.sync_copy(z16_hbm.at[pl.ds(s * stripe, stripe)],
                        acc.at[pl.ds(s * stripe, stripe)])
        plsc.subcore_barrier()

        @pl.loop(0, ch, step=4)
        def _(j):
            ds = [pltpu.async_copy(ones_v, acc.at[dst_v.at[j + b]],
                                   ssem, add=True) for b in range(4)]
            for dsc in ds:
                dsc.wait()

        plsc.subcore_barrier()
        pltpu.sync_copy(acc.at[pl.ds(s * stripe, stripe)],
                        out_hbm.at[pl.ds(c * n_pad + s * stripe, stripe)])

    return deg_kernel


def _layer_kernel(n_pad, ch, d):
    assert ch % (2 * NBUF) == 0
    stripe = n_pad // NS
    nblk = ch // NBUF
    mesh = plsc.VectorSubcoreMesh(core_axis_name="c", subcore_axis_name="s", num_cores=NC, num_subcores=NS)

    @functools.partial(
        pl.kernel,
        out_type=jax.ShapeDtypeStruct((NC * n_pad, d), jnp.float32),
        mesh=mesh,
        scratch_types=(
            [pltpu.VMEM((ch, CHUNK), jnp.int32),
             pltpu.VMEM((2 * NBUF, CHUNK), jnp.int32)]
            + [pltpu.VMEM((CHUNK, d), jnp.float32)] * NBUF
            + [pltpu.VMEM_SHARED((n_pad, d), jnp.float32)]
            + [pltpu.SemaphoreType.DMA] * 3
        ),
    )
    def layer_kernel(h_hbm, src_hbm, dst_hbm, z_hbm, out_hbm,
                     src_v, dstR, *rest):
        bufs = rest[:NBUF]
        acc = rest[NBUF]
        gsem, ssem, isem = rest[NBUF + 1:]
        c = lax.axis_index("c")
        s = lax.axis_index("s")
        wid = c * NS + s

        def wait_idx():
            pltpu.make_async_copy(
                dst_hbm.at[wid, pl.ds(0, NBUF)],
                dstR.at[pl.ds(0, NBUF)], isem).wait()

        pltpu.sync_copy(z_hbm.at[pl.ds(s * stripe, stripe)],
                        acc.at[pl.ds(s * stripe, stripe)])
        pltpu.sync_copy(src_hbm.at[wid], src_v)
        pltpu.sync_copy(dst_hbm.at[wid, pl.ds(0, NBUF)],
                        dstR.at[pl.ds(0, NBUF)])
        plsc.subcore_barrier()
        pltpu.async_copy(dst_hbm.at[wid, pl.ds(NBUF, NBUF)],
                         dstR.at[pl.ds(NBUF, NBUF)], isem)

        def phase(i, cur, nxt):
            j = i * NBUF
            for b in range(NBUF):
                pltpu.sync_copy(h_hbm.at[src_v.at[j + b]], bufs[b])
                pltpu.sync_copy(bufs[b], acc.at[dstR.at[cur * NBUF + b]],
                                add=True)
            wait_idx()
            blk = jnp.minimum(i + 2, nblk - 1)
            pltpu.async_copy(dst_hbm.at[wid, pl.ds(blk * NBUF, NBUF)],
                             dstR.at[pl.ds(cur * NBUF, NBUF)], isem)

        @pl.loop(0, nblk // 2)
        def _(k):
            phase(2 * k, 0, 1)
            phase(2 * k + 1, 1, 0)

        wait_idx()
        plsc.subcore_barrier()
        pltpu.sync_copy(acc.at[pl.ds(s * stripe, stripe)],
                        out_hbm.at[pl.ds(c * n_pad + s * stripe, stripe)])

    return layer_kernel


def _dis_from_degp(degp, n_pad):
    deg = degp[:n_pad, 0:1] + degp[n_pad:, 0:1]
    return jnp.where(deg > 0.0, lax.rsqrt(deg), 0.0)


def _prep_body(n_pad, x_ref, degp_ref, w_ref, out_ref):
    dis = _dis_from_degp(degp_ref[...], n_pad)
    h = jnp.dot(x_ref[...], w_ref[...], preferred_element_type=jnp.float32)
    out_ref[...] = h * dis


def _mid_body(n_pad, p_ref, degp_ref, b_ref, w_ref, out_ref):
    dis = _dis_from_degp(degp_ref[...], n_pad)
    p = p_ref[...]
    acc = p[:n_pad] + p[n_pad:]
    t = jnp.maximum(acc * dis + b_ref[...], 0.0)
    out_ref[...] = jnp.dot(
        t, w_ref[...], preferred_element_type=jnp.float32) * dis


def _final_body(n_pad, p_ref, degp_ref, b_ref, wh_ref, bh_ref,
                emb_ref, pred_ref):
    dis = _dis_from_degp(degp_ref[...], n_pad)
    p = p_ref[...]
    acc = p[:n_pad] + p[n_pad:]
    emb = jnp.maximum(acc * dis + b_ref[...], 0.0)
    pred = jnp.dot(
        emb, wh_ref[...], preferred_element_type=jnp.float32) + bh_ref[...]
    emb_ref[...] = jnp.nan_to_num(emb)
    pred_ref[...] = jnp.nan_to_num(pred)


def kernel(x, edge_index, W1, b1, W2, b2, W3, b3, Wh, bh):
    x = x.astype(jnp.float32)
    n, d = x.shape
    e = edge_index.shape[1]
    n_pad = ((n + CHUNK - 1) // CHUNK) * CHUNK
    e_tot = e + n
    ch = (2 * NBUF) * (-(-e_tot // (NW * CHUNK * 2 * NBUF)))
    e_pad = NW * ch * CHUNK

    loops = jnp.arange(n, dtype=edge_index.dtype)
    fill = jnp.full((e_pad - e_tot,), n, dtype=edge_index.dtype)
    src = jnp.concatenate([edge_index[0], loops, fill]).reshape(NW, ch, CHUNK)
    dst = jnp.concatenate([edge_index[1], loops, fill]).reshape(NW, ch, CHUNK)

    x_pad = jnp.zeros((n_pad, d), jnp.float32).at[:n].set(x)
    z = jnp.zeros((n_pad, d), jnp.float32)
    ones = jnp.ones((CHUNK, d), jnp.float32)
    b1r = b1.reshape(1, d)
    b2r = b2.reshape(1, d)
    b3r = b3.reshape(1, d)
    bhr = bh.reshape(1, 1)

    deg_k = _deg_kernel(n_pad, ch, d)
    lay_k = _layer_kernel(n_pad, ch, d)

    prep = pl.pallas_call(
        functools.partial(_prep_body, n_pad),
        out_shape=jax.ShapeDtypeStruct((n_pad, d), jnp.float32))
    mid = pl.pallas_call(
        functools.partial(_mid_body, n_pad),
        out_shape=jax.ShapeDtypeStruct((n_pad, d), jnp.float32))
    final = pl.pallas_call(
        functools.partial(_final_body, n_pad),
        out_shape=(jax.ShapeDtypeStruct((n_pad, d), jnp.float32),
                   jax.ShapeDtypeStruct((n_pad, 1), jnp.float32)))

    degp = deg_k(dst, ones, z)
    h = prep(x_pad, degp, W1)
    p = lay_k(h, src, dst, z)
    h = mid(p, degp, b1r, W2)
    p = lay_k(h, src, dst, z)
    h = mid(p, degp, b2r, W3)
    p = lay_k(h, src, dst, z)
    emb, pred = final(p, degp, b3r, Wh, bhr)
    return emb[:n], pred[:n, 0]


# speedup vs baseline: 3.4433x; 3.4433x over previous
import functools

import jax
import jax.numpy as jnp
from jax import lax
from jax.experimental import pallas as pl
from jax.experimental.pallas import tpu as pltpu
from jax.experimental.pallas import tpu_sc as plsc

NC = 2
NS = 16
NW = NC * NS
CHUNK = 128


def _deg_kernel(n_pad, ch, d):
    stripe = n_pad // NS
    mesh = plsc.VectorSubcoreMesh(core_axis_name="c", subcore_axis_name="s",
                                  num_cores=NC, num_subcores=NS)

    @functools.partial(
        pl.kernel,
        out_type=jax.ShapeDtypeStruct((NC * n_pad, d), jnp.float32),
        mesh=mesh,
        scratch_types=[
            pltpu.VMEM((ch, CHUNK), jnp.int32),
            pltpu.VMEM((CHUNK, d), jnp.float32),
            pltpu.VMEM_SHARED((n_pad, d), jnp.float32),
        ],
    )
    def deg_kernel(dst_hbm, ones_hbm, z_hbm, out_hbm, dst_v, ones_v, acc):
        c = lax.axis_index("c")
        s = lax.axis_index("s")
        wid = c * NS + s
        pltpu.sync_copy(dst_hbm.at[wid], dst_v)
        pltpu.sync_copy(ones_hbm, ones_v)
        pltpu.sync_copy(z_hbm.at[pl.ds(s * stripe, stripe)],
                        acc.at[pl.ds(s * stripe, stripe)])
        plsc.subcore_barrier()

        @pl.loop(0, ch)
        def _(j):
            pltpu.sync_copy(ones_v, acc.at[dst_v.at[j]], add=True)

        plsc.subcore_barrier()
        pltpu.sync_copy(acc.at[pl.ds(s * stripe, stripe)],
                        out_hbm.at[pl.ds(c * n_pad + s * stripe, stripe)])

    return deg_kernel


def _layer_kernel(n_pad, ch, d):
    stripe = n_pad // NS
    mesh = plsc.VectorSubcoreMesh(core_axis_name="c", subcore_axis_name="s",
                                  num_cores=NC, num_subcores=NS)

    @functools.partial(
        pl.kernel,
        out_type=jax.ShapeDtypeStruct((NC * n_pad, d), jnp.float32),
        mesh=mesh,
        scratch_types=[
            pltpu.VMEM((ch, CHUNK), jnp.int32),
            pltpu.VMEM((ch, CHUNK), jnp.int32),
            pltpu.VMEM((CHUNK, d), jnp.float32),
            pltpu.VMEM_SHARED((n_pad, d), jnp.float32),
        ],
    )
    def layer_kernel(h_hbm, src_hbm, dst_hbm, z_hbm, out_hbm,
                     src_v, dst_v, gbuf, acc):
        c = lax.axis_index("c")
        s = lax.axis_index("s")
        wid = c * NS + s
        pltpu.sync_copy(src_hbm.at[wid], src_v)
        pltpu.sync_copy(dst_hbm.at[wid], dst_v)
        pltpu.sync_copy(z_hbm.at[pl.ds(s * stripe, stripe)],
                        acc.at[pl.ds(s * stripe, stripe)])
        plsc.subcore_barrier()

        @pl.loop(0, ch)
        def _(j):
            pltpu.sync_copy(h_hbm.at[src_v.at[j]], gbuf)
            pltpu.sync_copy(gbuf, acc.at[dst_v.at[j]], add=True)

        plsc.subcore_barrier()
        pltpu.sync_copy(acc.at[pl.ds(s * stripe, stripe)],
                        out_hbm.at[pl.ds(c * n_pad + s * stripe, stripe)])

    return layer_kernel


def _dis_from_degp(degp, n_pad):
    deg = degp[:n_pad, 0:1] + degp[n_pad:, 0:1]
    return jnp.where(deg > 0.0, lax.rsqrt(deg), 0.0)


def _prep_body(n_pad, x_ref, degp_ref, w_ref, out_ref):
    dis = _dis_from_degp(degp_ref[...], n_pad)
    h = jnp.dot(x_ref[...], w_ref[...], preferred_element_type=jnp.float32)
    out_ref[...] = h * dis


def _mid_body(n_pad, p_ref, degp_ref, b_ref, w_ref, out_ref):
    dis = _dis_from_degp(degp_ref[...], n_pad)
    p = p_ref[...]
    acc = p[:n_pad] + p[n_pad:]
    t = jnp.maximum(acc * dis + b_ref[...], 0.0)
    out_ref[...] = jnp.dot(
        t, w_ref[...], preferred_element_type=jnp.float32) * dis


def _final_body(n_pad, p_ref, degp_ref, b_ref, wh_ref, bh_ref,
                emb_ref, pred_ref):
    dis = _dis_from_degp(degp_ref[...], n_pad)
    p = p_ref[...]
    acc = p[:n_pad] + p[n_pad:]
    emb = jnp.maximum(acc * dis + b_ref[...], 0.0)
    pred = jnp.dot(
        emb, wh_ref[...], preferred_element_type=jnp.float32) + bh_ref[...]
    emb_ref[...] = jnp.nan_to_num(emb)
    pred_ref[...] = jnp.nan_to_num(pred)


def kernel(x, edge_index, W1, b1, W2, b2, W3, b3, Wh, bh):
    x = x.astype(jnp.float32)
    n, d = x.shape
    e = edge_index.shape[1]
    n_pad = ((n + NS * CHUNK - 1) // (NS * CHUNK)) * (NS * CHUNK)
    e_tot = e + n
    ch = -(-e_tot // (NW * CHUNK))
    e_pad = NW * ch * CHUNK

    loops = jnp.arange(n, dtype=edge_index.dtype)
    fill = jnp.full((e_pad - e_tot,), n, dtype=edge_index.dtype)
    src = jnp.concatenate([edge_index[0], loops, fill]).reshape(NW, ch, CHUNK)
    dst = jnp.concatenate([edge_index[1], loops, fill]).reshape(NW, ch, CHUNK)

    x_pad = jnp.zeros((n_pad, d), jnp.float32).at[:n].set(x)
    z = jnp.zeros((n_pad, d), jnp.float32)
    ones = jnp.ones((CHUNK, d), jnp.float32)
    b1r = b1.reshape(1, d)
    b2r = b2.reshape(1, d)
    b3r = b3.reshape(1, d)
    bhr = bh.reshape(1, 1)

    deg_k = _deg_kernel(n_pad, ch, d)
    lay_k = _layer_kernel(n_pad, ch, d)

    prep = pl.pallas_call(
        functools.partial(_prep_body, n_pad),
        out_shape=jax.ShapeDtypeStruct((n_pad, d), jnp.float32))
    mid = pl.pallas_call(
        functools.partial(_mid_body, n_pad),
        out_shape=jax.ShapeDtypeStruct((n_pad, d), jnp.float32))
    final = pl.pallas_call(
        functools.partial(_final_body, n_pad),
        out_shape=(jax.ShapeDtypeStruct((n_pad, d), jnp.float32),
                   jax.ShapeDtypeStruct((n_pad, 1), jnp.float32)))

    degp = deg_k(dst, ones, z)
    h = prep(x_pad, degp, W1)
    p = lay_k(h, src, dst, z)
    h = mid(p, degp, b1r, W2)
    p = lay_k(h, src, dst, z)
    h = mid(p, degp, b2r, W3)
    p = lay_k(h, src, dst, z)
    emb, pred = final(p, degp, b3r, Wh, bhr)
    return emb[:n], pred[:n, 0]
